# Initial kernel scaffold; baseline (speedup 1.0000x reference)
import jax
import jax.numpy as jnp
from jax.experimental import pallas as pl


def kernel(centers):
    raise NotImplementedError("write your pallas kernel here")


# speedup vs baseline: 12.6861x; 12.6861x over previous
import functools

import jax
import jax.numpy as jnp
import numpy as np
from jax.experimental import pallas as pl

_MASK_RATIO = 0.6
_BLOCK_RATIO = 0.02
_ADJUST_RATIO = 0.1

_consts_cache = {}


def _get_consts(B, P):
    if (B, P) in _consts_cache:
        return _consts_cache[(B, P)]
    block_size = int(_BLOCK_RATIO * P)
    block_fraction = (_MASK_RATIO - _ADJUST_RATIO) / block_size
    num_centers = round(P * block_fraction)
    with jax.ensure_compile_time_eval():
        k1, k2 = jax.random.split(jax.random.key(42))
        ru1 = np.asarray(jax.random.uniform(k1, (B, P), dtype=jnp.float32))
        ru2 = np.asarray(jax.random.uniform(k2, (B, P), dtype=jnp.float32))
    ci = np.argsort(ru1, axis=-1, kind="stable")[:, :num_centers].astype(np.int32)
    perm = np.argsort(ru2, axis=-1, kind="stable")
    rk = np.empty((B, P), np.int32)
    rk[np.arange(B)[:, None], perm] = np.arange(P, dtype=np.int32)[None, :]
    _consts_cache[(B, P)] = (ci, rk)
    return ci, rk


def _body(pts_ref, sel_ref, rk_ref, out_ref, *, NC, K, NM, P):
    p = pts_ref[0]
    sel = sel_ref[0]
    rk = rk_ref[0]

    px, py, pz = p[0:1, :], p[1:2, :], p[2:3, :]
    s2 = px * px + py * py + pz * pz
    sx, sy, sz = sel[:, 0:1], sel[:, 1:2], sel[:, 2:3]
    s1 = sx * sx + sy * sy + sz * sz
    bf = lambda v: v.astype(jnp.bfloat16).astype(jnp.float32)
    dot = bf(sx) * bf(px) + bf(sy) * bf(py) + bf(sz) * bf(pz)
    d = (s1 + s2) - 2.0 * dot
    d = jnp.maximum(d, 0.0)
    keys = jax.lax.bitcast_convert_type(d, jnp.int32)

    kf = jnp.float32(K)

    def tbody(_, lh):
        lo, hi = lh
        mid = lo + ((hi - lo) >> 1)
        cnt = jnp.sum(jnp.where(keys <= mid, 1.0, 0.0), axis=1, keepdims=True)
        ge = cnt >= kf
        return jnp.where(ge, lo, mid + 1), jnp.where(ge, mid, hi)

    lo0 = jnp.zeros((NC, 1), jnp.int32)
    hi0 = jnp.full((NC, 1), jnp.int32(0x7F7FFFFF))
    t, _ = jax.lax.fori_loop(0, 31, tbody, (lo0, hi0))

    n_less = jnp.sum(jnp.where(keys < t, 1.0, 0.0), axis=1, keepdims=True)
    extra = kf - n_less
    eq = keys == t
    jidx = jax.lax.broadcasted_iota(jnp.int32, (NC, P), 1)

    def jbody(_, lh):
        lo, hi = lh
        mid = lo + ((hi - lo) >> 1)
        cnt = jnp.sum(jnp.where(eq & (jidx <= mid), 1.0, 0.0),
                      axis=1, keepdims=True)
        ge = cnt >= extra
        return jnp.where(ge, lo, mid + 1), jnp.where(ge, mid, hi)

    jlo0 = jnp.zeros((NC, 1), jnp.int32)
    jhi0 = jnp.full((NC, 1), jnp.int32(P - 1))
    jthr, _ = jax.lax.fori_loop(0, 13, jbody, (jlo0, jhi0))

    covered = (keys < t) | (eq & (jidx <= jthr))
    cov = jnp.any(covered, axis=0, keepdims=True)

    T = jnp.sum(jnp.where(cov, 1.0, 0.0), axis=1, keepdims=True)
    A = jnp.float32(NM) - T

    uncov = ~cov

    def fbody(_, lh):
        lo, hi = lh
        mid = lo + ((hi - lo) >> 1)
        cnt = jnp.sum(jnp.where(uncov & (rk <= mid), 1.0, 0.0),
                      axis=1, keepdims=True)
        ge = cnt >= A
        return jnp.where(ge, lo, mid + 1), jnp.where(ge, mid, hi)

    flo0 = jnp.zeros((1, 1), jnp.int32)
    fhi0 = jnp.full((1, 1), jnp.int32(P - 1))
    m, _ = jax.lax.fori_loop(0, 13, fbody, (flo0, fhi0))

    out_ref[0] = (cov | (uncov & (rk <= m))).astype(jnp.int32)


def kernel(centers):
    B, P, _ = centers.shape
    num_masks = round(_MASK_RATIO * P)
    block_size = int(_BLOCK_RATIO * P)
    num_centers = round(P * (_MASK_RATIO - _ADJUST_RATIO) / block_size)
    ci, rk = _get_consts(B, P)

    sel = jnp.take_along_axis(centers, jnp.asarray(ci)[..., None], axis=1)
    nc_pad = -(-num_centers // 8) * 8
    if nc_pad != num_centers:
        sel = jnp.concatenate(
            [sel, jnp.broadcast_to(sel[:, :1, :],
                                   (B, nc_pad - num_centers, 3))], axis=1)
    pts = centers.transpose(0, 2, 1)
    rk3 = jnp.asarray(rk).reshape(B, 1, P)

    body = functools.partial(_body, NC=nc_pad, K=block_size,
                             NM=num_masks, P=P)
    out = pl.pallas_call(
        body,
        grid=(B,),
        in_specs=[
            pl.BlockSpec((1, 3, P), lambda b: (b, 0, 0)),
            pl.BlockSpec((1, nc_pad, 3), lambda b: (b, 0, 0)),
            pl.BlockSpec((1, 1, P), lambda b: (b, 0, 0)),
        ],
        out_specs=pl.BlockSpec((1, 1, P), lambda b: (b, 0, 0)),
        out_shape=jax.ShapeDtypeStruct((B, 1, P), jnp.int32),
    )(pts, sel, rk3)
    return out.reshape(B, P).astype(bool)


# speedup vs baseline: 16.1193x; 1.2706x over previous
import functools

import jax
import jax.numpy as jnp
import numpy as np
from jax.experimental import pallas as pl

_MASK_RATIO = 0.6
_BLOCK_RATIO = 0.02
_ADJUST_RATIO = 0.1

_consts_cache = {}


def _get_consts(B, P):
    if (B, P) in _consts_cache:
        return _consts_cache[(B, P)]
    block_size = int(_BLOCK_RATIO * P)
    block_fraction = (_MASK_RATIO - _ADJUST_RATIO) / block_size
    num_centers = round(P * block_fraction)
    with jax.ensure_compile_time_eval():
        k1, k2 = jax.random.split(jax.random.key(42))
        ru1 = np.asarray(jax.random.uniform(k1, (B, P), dtype=jnp.float32))
        ru2 = np.asarray(jax.random.uniform(k2, (B, P), dtype=jnp.float32))
    ci = np.argsort(ru1, axis=-1, kind="stable")[:, :num_centers].astype(np.int32)
    perm = np.argsort(ru2, axis=-1, kind="stable")
    rk = np.empty((B, P), np.int32)
    rk[np.arange(B)[:, None], perm] = np.arange(P, dtype=np.int32)[None, :]
    _consts_cache[(B, P)] = (ci, rk)
    return ci, rk


def _body(pts_ref, sel_ref, rk_ref, out_ref, *, NC, K, NM, P):
    p = pts_ref[0]
    sel = sel_ref[0]
    rk = rk_ref[0]

    px, py, pz = p[0:1, :], p[1:2, :], p[2:3, :]
    s2 = px * px + py * py + pz * pz
    sx, sy, sz = sel[:, 0:1], sel[:, 1:2], sel[:, 2:3]
    s1 = sx * sx + sy * sy + sz * sz
    bf = lambda v: v.astype(jnp.bfloat16).astype(jnp.float32)
    dot = bf(sx) * bf(px) + bf(sy) * bf(py) + bf(sz) * bf(pz)
    d = (s1 + s2) - 2.0 * dot
    d = jnp.maximum(d, 0.0)
    keys = jax.lax.bitcast_convert_type(d, jnp.int32)

    kf = jnp.float32(K)

    def tbody(_, lh):
        lo, hi = lh
        mid = lo + ((hi - lo) >> 1)
        cnt = jnp.sum(jnp.where(keys <= mid, 1.0, 0.0), axis=1, keepdims=True)
        ge = cnt >= kf
        return jnp.where(ge, lo, mid + 1), jnp.where(ge, mid, hi)

    lo0 = jnp.zeros((NC, 1), jnp.int32)
    hi0 = jnp.full((NC, 1), jnp.int32(0x7F7FFFFF))
    t, _ = jax.lax.fori_loop(0, 31, tbody, (lo0, hi0))

    n_less = jnp.sum(jnp.where(keys < t, 1.0, 0.0), axis=1, keepdims=True)
    extra = kf - n_less
    eq = keys == t
    n_eq = jnp.sum(jnp.where(eq, 1.0, 0.0), axis=1, keepdims=True)
    jidx = jax.lax.broadcasted_iota(jnp.int32, (NC, P), 1)

    def jsearch():
        def jbody(_, lh):
            lo, hi = lh
            mid = lo + ((hi - lo) >> 1)
            cnt = jnp.sum(jnp.where(eq & (jidx <= mid), 1.0, 0.0),
                          axis=1, keepdims=True)
            ge = cnt >= extra
            return jnp.where(ge, lo, mid + 1), jnp.where(ge, mid, hi)

        jlo0 = jnp.zeros((NC, 1), jnp.int32)
        jhi0 = jnp.full((NC, 1), jnp.int32(P - 1))
        lo, _ = jax.lax.fori_loop(0, 13, jbody, (jlo0, jhi0))
        return lo

    tie_any = jnp.any(n_eq > extra)
    jthr = jax.lax.cond(tie_any, jsearch,
                        lambda: jnp.full((NC, 1), jnp.int32(P - 1)))

    covered = (keys < t) | (eq & (jidx <= jthr))
    cov = jnp.any(covered, axis=0, keepdims=True)

    T = jnp.sum(jnp.where(cov, 1.0, 0.0), axis=1, keepdims=True)
    A = jnp.float32(NM) - T

    uncov = ~cov

    def fbody(_, lh):
        lo, hi = lh
        mid = lo + ((hi - lo) >> 1)
        cnt = jnp.sum(jnp.where(uncov & (rk <= mid), 1.0, 0.0),
                      axis=1, keepdims=True)
        ge = cnt >= A
        return jnp.where(ge, lo, mid + 1), jnp.where(ge, mid, hi)

    flo0 = jnp.zeros((1, 1), jnp.int32)
    fhi0 = jnp.full((1, 1), jnp.int32(P - 1))
    m, _ = jax.lax.fori_loop(0, 13, fbody, (flo0, fhi0))

    out_ref[0] = (cov | (uncov & (rk <= m))).astype(jnp.int32)


def kernel(centers):
    B, P, _ = centers.shape
    num_masks = round(_MASK_RATIO * P)
    block_size = int(_BLOCK_RATIO * P)
    num_centers = round(P * (_MASK_RATIO - _ADJUST_RATIO) / block_size)
    ci, rk = _get_consts(B, P)

    sel = jnp.take_along_axis(centers, jnp.asarray(ci)[..., None], axis=1)
    nc_pad = -(-num_centers // 8) * 8
    if nc_pad != num_centers:
        sel = jnp.concatenate(
            [sel, jnp.broadcast_to(sel[:, :1, :],
                                   (B, nc_pad - num_centers, 3))], axis=1)
    pts = centers.transpose(0, 2, 1)
    rk3 = jnp.asarray(rk).reshape(B, 1, P)

    body = functools.partial(_body, NC=nc_pad, K=block_size,
                             NM=num_masks, P=P)
    out = pl.pallas_call(
        body,
        grid=(B,),
        in_specs=[
            pl.BlockSpec((1, 3, P), lambda b: (b, 0, 0)),
            pl.BlockSpec((1, nc_pad, 3), lambda b: (b, 0, 0)),
            pl.BlockSpec((1, 1, P), lambda b: (b, 0, 0)),
        ],
        out_specs=pl.BlockSpec((1, 1, P), lambda b: (b, 0, 0)),
        out_shape=jax.ShapeDtypeStruct((B, 1, P), jnp.int32),
    )(pts, sel, rk3)
    return out.reshape(B, P).astype(bool)
